# Initial kernel scaffold; baseline (speedup 1.0000x reference)
import jax
import jax.numpy as jnp
from jax.experimental import pallas as pl


def kernel(x, edge_index, W0, a0, W1, a1, W2, a2):
    raise NotImplementedError("write your pallas kernel here")


# speedup vs baseline: 1.0095x; 1.0095x over previous
import functools

import jax
import jax.numpy as jnp
from jax.experimental import pallas as pl

N = 10000
E = 320000
IN = 128
HID = 128
OUT = 128
H = 8

_BLK = 512
_NPAD = 10240


def _mm_kernel(x_ref, w_ref, o_ref, *, elu_in: bool):
    x = x_ref[...]
    if elu_in:
        x = jnp.where(x > 0, x, jnp.exp(x) - 1.0)
    o_ref[...] = jnp.dot(x, w_ref[...], preferred_element_type=jnp.float32)


def _matmul(x, w, elu_in: bool):
    npad, k = x.shape
    m = w.shape[1]
    grid = (npad // _BLK,)
    return pl.pallas_call(
        functools.partial(_mm_kernel, elu_in=elu_in),
        grid=grid,
        in_specs=[
            pl.BlockSpec((_BLK, k), lambda i: (i, 0)),
            pl.BlockSpec((k, m), lambda i: (0, 0)),
        ],
        out_specs=pl.BlockSpec((_BLK, m), lambda i: (i, 0)),
        out_shape=jax.ShapeDtypeStruct((npad, m), jnp.float32),
    )(x, w)


def _edge_stage(hcat, src, dst, a):
    nh = a.shape[0]
    h = hcat.reshape(N, nh, HID)
    hs = h[src]
    hd = h[dst]
    m = hs + hd
    m = jnp.maximum(m, 0.2 * m)
    e = jnp.einsum('enk,nk->en', m, a)
    emax = jax.ops.segment_max(e, dst, num_segments=N)
    emax = jnp.where(jnp.isfinite(emax), emax, 0.0)
    ex = jnp.exp(e - emax[dst])
    den = jax.ops.segment_sum(ex, dst, num_segments=N)
    alpha = ex / (den[dst] + 1e-16)
    out = jax.ops.segment_sum(alpha[:, :, None] * hs, dst, num_segments=N)
    return out.reshape(N, nh * HID)


def _pool_kernel(h_ref, o_ref, m_ref):
    i = pl.program_id(0)
    blkmax = jnp.max(h_ref[...], axis=0, keepdims=True)

    @pl.when(i == 0)
    def _():
        m_ref[...] = blkmax

    @pl.when(i > 0)
    def _():
        m_ref[...] = jnp.maximum(m_ref[...], blkmax)

    o_ref[...] = h_ref[...]


def kernel(x, edge_index, W0, a0, W1, a1, W2, a2):
    src = edge_index[0]
    dst = edge_index[1]

    w0 = jnp.transpose(W0, (1, 0, 2)).reshape(IN, H * HID)
    w1 = jnp.transpose(W1, (1, 0, 2)).reshape(H * HID, H * HID)

    xp = jnp.pad(x, ((0, _NPAD - N), (0, 0)))

    h = _matmul(xp, w0, elu_in=False)[:N]
    h = _edge_stage(h, src, dst, a0)
    hp = jnp.pad(h, ((0, _NPAD - N), (0, 0)))
    h = _matmul(hp, w1, elu_in=True)[:N]
    h = _edge_stage(h, src, dst, a1)
    hp = jnp.pad(h, ((0, _NPAD - N), (0, 0)))
    h = _matmul(hp, W2, elu_in=True)[:N]
    h = _edge_stage(h, src, dst, a2[None])

    hp = jnp.pad(h, ((0, _NPAD - N), (0, 0)), constant_values=-jnp.inf)
    hcopy, logits = pl.pallas_call(
        _pool_kernel,
        grid=(_NPAD // _BLK,),
        in_specs=[pl.BlockSpec((_BLK, OUT), lambda i: (i, 0))],
        out_specs=[
            pl.BlockSpec((_BLK, OUT), lambda i: (i, 0)),
            pl.BlockSpec((1, OUT), lambda i: (0, 0)),
        ],
        out_shape=[
            jax.ShapeDtypeStruct((_NPAD, OUT), jnp.float32),
            jax.ShapeDtypeStruct((1, OUT), jnp.float32),
        ],
    )(hp)
    h = hcopy[:N]
    cat = jnp.concatenate([h, jnp.broadcast_to(logits, h.shape)], axis=1)
    return (cat, logits)
